# baseline (speedup 1.0000x reference)
import functools

import jax
import jax.numpy as jnp
from jax import lax
from jax.experimental import pallas as pl
from jax.experimental.pallas import tpu as pltpu
from jax.experimental.pallas import tpu_sc as plsc

_NUM_CORES = 2


@functools.cache
def _sc_gather(t: int, embed: int):
    mesh = plsc.VectorSubcoreMesh(core_axis_name="c", subcore_axis_name="s")

    @pl.kernel(
        out_type=jax.ShapeDtypeStruct((t, embed), jnp.float32),
        mesh=mesh,
        compiler_params=pltpu.CompilerParams(use_tc_tiling_on_sc=False),
        scratch_types=[
            pltpu.VMEM((t,), jnp.int32),
            pltpu.VMEM((t, embed), jnp.float32),
            pltpu.SemaphoreType.DMA,
        ],
    )
    def gather_kernel(idx_hbm, table_hbm, out_hbm, idx_v, rows_v, gsem):
        wid = lax.axis_index("s") * _NUM_CORES + lax.axis_index("c")

        @pl.when(wid == 0)
        def _():
            pltpu.sync_copy(idx_hbm, idx_v)
            pltpu.async_copy(table_hbm.at[idx_v], rows_v, gsem).wait()
            pltpu.sync_copy(rows_v, out_hbm)

    return gather_kernel


@functools.cache
def _tc_bcast(batch: int, t: int, embed: int, tblk: int):
    def body(emb_ref, out_ref):
        blk = emb_ref[0]
        out_ref[...] = jnp.broadcast_to(blk[:, :, None], (tblk, embed, batch))

    return pl.pallas_call(
        body,
        grid=(t // tblk,),
        in_specs=[pl.BlockSpec((1, tblk, embed), lambda i: (i, 0, 0))],
        out_specs=pl.BlockSpec((tblk, embed, batch), lambda i: (i, 0, 0)),
        out_shape=jax.ShapeDtypeStruct((t, embed, batch), jnp.float32),
    )


def kernel(x, x_index, table):
    batch, t = x.shape
    rows, embed = table.shape
    emb = _sc_gather(t, embed)(x_index.astype(jnp.int32), table)
    out_teb = _tc_bcast(batch, t, embed, 4)(emb.reshape(t // 4, 4, embed))
    return jnp.transpose(out_teb, (2, 0, 1))


# speedup vs baseline: 1.5850x; 1.5850x over previous
import functools

import jax
import jax.numpy as jnp
from jax import lax
from jax.experimental import pallas as pl
from jax.experimental.pallas import tpu as pltpu


@functools.cache
def _tc_bcast(batch: int, t: int, rows: int, embed: int, tblk: int):
    def body(idx_ref, table_ref, out_ref):
        i = pl.program_id(0)
        riota = lax.broadcasted_iota(jnp.int32, (rows, embed), 0)
        parts = []
        for r in range(tblk):
            s = idx_ref[i * tblk + r]
            parts.append(
                jnp.sum(
                    jnp.where(riota == s, table_ref[...], 0.0),
                    axis=0,
                    keepdims=True,
                )
            )
        blk = jnp.concatenate(parts, axis=0)
        out_ref[...] = jnp.broadcast_to(blk[:, :, None], (tblk, embed, batch))

    return pl.pallas_call(
        body,
        grid=(t // tblk,),
        in_specs=[
            pl.BlockSpec(memory_space=pltpu.SMEM),
            pl.BlockSpec((rows, embed), lambda i: (0, 0)),
        ],
        out_specs=pl.BlockSpec((tblk, embed, batch), lambda i: (i, 0, 0)),
        out_shape=jax.ShapeDtypeStruct((t, embed, batch), jnp.float32),
    )


def kernel(x, x_index, table):
    batch, t = x.shape
    rows, embed = table.shape
    out_teb = _tc_bcast(batch, t, rows, embed, 4)(
        x_index.astype(jnp.int32), table
    )
    return jnp.transpose(out_teb, (2, 0, 1))
